# baseline (speedup 1.0000x reference)
import functools

import jax
import jax.numpy as jnp
import numpy as np
from jax import lax
from jax.experimental import pallas as pl
from jax.experimental.pallas import tpu as pltpu
from jax.experimental.pallas import tpu_sc as plsc

B = 1024
DIM = 256
NNZ = DIM * (DIM + 1) // 2
SEG_W = 32
SEGS_PER_MAT = DIM * DIM // SEG_W
NSEG = sum(8 - (r // 32) for r in range(DIM))
IDX_ROWS = NSEG // 128


def _off(i):
    return DIM * i - (i * (i - 1)) // 2


def _build_tables():
    seg_idx = []
    pos = {}
    for r in range(DIM):
        for s2 in range(r // 32, 8):
            pos[(r, s2)] = len(seg_idx)
            seg_idx.append(
                (r // 8) * 64 + (s2 // 4) * 32 + (r % 8) * 4 + (s2 % 4)
            )
    stores = []
    for r in range(DIM - 1, -1, -1):
        d, sd = r % 16, r // 16
        for s in range(2 * (r // 32), 16):
            dst = _off(r) - d if (s == sd and d > 0) else _off(r) + 16 * s - r
            stores.append((pos[(r, s // 2)], s % 2, dst))
    return np.array(seg_idx, np.int32).reshape(IDX_ROWS, 128), stores


_SEG_IDX, _STORES = _build_tables()


def kernel(input):
    x_tbl = (
        input.reshape(B, 32, 8, 2, 128)
        .transpose(0, 1, 3, 2, 4)
        .reshape(B * SEGS_PER_MAT, SEG_W)
    )
    seg_idx = jnp.asarray(_SEG_IDX)

    info = plsc.get_sparse_core_info()
    nc, ns = info.num_cores, info.num_subcores
    nw = nc * ns
    bpw = B // nw
    ngrp = bpw // 2

    mesh = plsc.VectorSubcoreMesh(core_axis_name="c", subcore_axis_name="s")

    @functools.partial(
        pl.kernel,
        mesh=mesh,
        out_type=jax.ShapeDtypeStruct((B * NNZ,), jnp.float32),
        scratch_types=[
            pltpu.VMEM((NSEG, SEG_W), jnp.float32),
            pltpu.VMEM((NSEG, SEG_W), jnp.float32),
            pltpu.VMEM((NNZ,), jnp.float32),
            pltpu.VMEM((IDX_ROWS, 128), jnp.int32),
            pltpu.SemaphoreType.DMA,
            pltpu.SemaphoreType.DMA,
            pltpu.SemaphoreType.DMA,
        ],
        compiler_params=pltpu.CompilerParams(use_tc_tiling_on_sc=False),
    )
    def tri_kernel(x_hbm, sidx_hbm, out_hbm, stage0_v, stage1_v, out_v,
                   idx_v, in0_sem, in1_sem, out_sem):
        wid = lax.axis_index("s") * nc + lax.axis_index("c")
        b0 = wid * bpw
        pltpu.sync_copy(sidx_hbm, idx_v)

        def fire_gathers(b, stage_v, sem):
            tbl_b = x_hbm.at[pl.ds(b * SEGS_PER_MAT, SEGS_PER_MAT)]
            for j in range(IDX_ROWS):
                pltpu.async_copy(
                    tbl_b.at[idx_v.at[j]],
                    stage_v.at[pl.ds(128 * j, 128)],
                    sem,
                )

        def drain_gathers(stage_v, sem):
            pltpu.make_async_copy(
                x_hbm.at[pl.ds(0, NSEG)], stage_v, sem
            ).wait()

        def wait_out():
            pltpu.make_async_copy(
                out_v, out_hbm.at[pl.ds(b0 * NNZ, NNZ)], out_sem
            ).wait()

        def compact_and_flush(b, stage_v):
            for k, half, dst in _STORES:
                out_v[pl.ds(dst, 16)] = stage_v[k, pl.ds(16 * half, 16)]
            pltpu.async_copy(out_v, out_hbm.at[pl.ds(b * NNZ, NNZ)], out_sem)

        fire_gathers(b0, stage0_v, in0_sem)

        def group_body(g, carry):
            be = b0 + 2 * g
            bo = be + 1
            bn = jnp.where(g + 1 < ngrp, be + 2, b0)

            fire_gathers(bo, stage1_v, in1_sem)
            drain_gathers(stage0_v, in0_sem)

            @pl.when(g > 0)
            def _():
                wait_out()

            compact_and_flush(be, stage0_v)

            fire_gathers(bn, stage0_v, in0_sem)
            drain_gathers(stage1_v, in1_sem)
            wait_out()
            compact_and_flush(bo, stage1_v)
            return carry

        lax.fori_loop(0, ngrp, group_body, 0)
        drain_gathers(stage0_v, in0_sem)
        wait_out()

    y = tri_kernel(x_tbl, seg_idx)
    return y.reshape(B, NNZ, 1)


# speedup vs baseline: 1.0742x; 1.0742x over previous
import functools

import jax
import jax.numpy as jnp
import numpy as np
from jax import lax
from jax.experimental import pallas as pl
from jax.experimental.pallas import tpu as pltpu
from jax.experimental.pallas import tpu_sc as plsc

B = 1024
DIM = 256
NNZ = DIM * (DIM + 1) // 2
SEG_W = 16
SEGS_PER_MAT = DIM * DIM // SEG_W
NSEG = sum(16 - (r // 16) for r in range(DIM))
IDX_ROWS = NSEG // 128


def _off(i):
    return DIM * i - (i * (i - 1)) // 2


def _build_tables():
    seg_idx = []
    pos = {}
    for r in range(DIM):
        for s in range(r // 16, 16):
            pos[(r, s)] = len(seg_idx)
            seg_idx.append((r // 8) * 128 + (s // 8) * 64 + (r % 8) * 8 + (s % 8))
    stores = []
    for r in range(DIM - 1, -1, -1):
        s0, d = r // 16, r % 16
        if d > 0:
            stores.append((pos[(r, s0)], _off(r) - d))
        for s in range(s0 + (1 if d > 0 else 0), 16):
            stores.append((pos[(r, s)], _off(r) + 16 * s - r))
    return np.array(seg_idx, np.int32).reshape(IDX_ROWS, 128), stores


_SEG_IDX, _STORES = _build_tables()


def kernel(input):
    x_tbl = (
        input.reshape(B, 32, 8, 2, 128)
        .transpose(0, 1, 3, 2, 4)
        .reshape(B * SEGS_PER_MAT, SEG_W)
    )
    seg_idx = jnp.asarray(_SEG_IDX)

    info = plsc.get_sparse_core_info()
    nc, ns = info.num_cores, info.num_subcores
    nw = nc * ns
    bpw = B // nw
    ngrp = bpw // 2

    mesh = plsc.VectorSubcoreMesh(core_axis_name="c", subcore_axis_name="s")

    @functools.partial(
        pl.kernel,
        mesh=mesh,
        out_type=jax.ShapeDtypeStruct((B * NNZ,), jnp.float32),
        scratch_types=[
            pltpu.VMEM((NSEG, SEG_W), jnp.float32),
            pltpu.VMEM((NSEG, SEG_W), jnp.float32),
            pltpu.VMEM((NNZ,), jnp.float32),
            pltpu.VMEM((IDX_ROWS, 128), jnp.int32),
            pltpu.SemaphoreType.DMA,
            pltpu.SemaphoreType.DMA,
            pltpu.SemaphoreType.DMA,
        ],
        compiler_params=pltpu.CompilerParams(use_tc_tiling_on_sc=False),
    )
    def tri_kernel(x_hbm, sidx_hbm, out_hbm, stage0_v, stage1_v, out_v,
                   idx_v, in0_sem, in1_sem, out_sem):
        wid = lax.axis_index("s") * nc + lax.axis_index("c")
        b0 = wid * bpw
        pltpu.sync_copy(sidx_hbm, idx_v)

        def fire_gathers(b, stage_v, sem):
            tbl_b = x_hbm.at[pl.ds(b * SEGS_PER_MAT, SEGS_PER_MAT)]
            for j in range(IDX_ROWS):
                pltpu.async_copy(
                    tbl_b.at[idx_v.at[j]],
                    stage_v.at[pl.ds(128 * j, 128)],
                    sem,
                )

        def drain_gathers(stage_v, sem):
            pltpu.make_async_copy(
                x_hbm.at[pl.ds(0, NSEG)], stage_v, sem
            ).wait()

        def wait_out():
            pltpu.make_async_copy(
                out_v, out_hbm.at[pl.ds(b0 * NNZ, NNZ)], out_sem
            ).wait()

        def compact_and_flush(b, stage_v):
            for k, dst in _STORES:
                out_v[pl.ds(dst, 16)] = stage_v[k, :]
            pltpu.async_copy(out_v, out_hbm.at[pl.ds(b * NNZ, NNZ)], out_sem)

        fire_gathers(b0, stage0_v, in0_sem)

        def group_body(g, carry):
            be = b0 + 2 * g
            bo = be + 1

            fire_gathers(bo, stage1_v, in1_sem)
            drain_gathers(stage0_v, in0_sem)

            @pl.when(g > 0)
            def _():
                wait_out()

            compact_and_flush(be, stage0_v)

            @pl.when(g + 1 < ngrp)
            def _():
                fire_gathers(be + 2, stage0_v, in0_sem)

            drain_gathers(stage1_v, in1_sem)
            wait_out()
            compact_and_flush(bo, stage1_v)
            return carry

        lax.fori_loop(0, ngrp, group_body, 0)
        wait_out()

    y = tri_kernel(x_tbl, seg_idx)
    return y.reshape(B, NNZ, 1)
